# Initial kernel scaffold; baseline (speedup 1.0000x reference)
import jax
import jax.numpy as jnp
from jax.experimental import pallas as pl


def kernel(x, router_w, w1, w2, w3):
    raise NotImplementedError("write your pallas kernel here")


# speedup vs baseline: 1.6922x; 1.6922x over previous
import functools

import jax
import jax.numpy as jnp
from jax import lax
from jax.experimental import pallas as pl
from jax.experimental.pallas import tpu as pltpu
from jax.experimental.pallas import tpu_sc as plsc

T = 2048
D = 1024
F = 4096
E = 8
BM = 256
NB = T * 2 // BM + E
NPAD = NB * BM
NW = 32
TPW = T // NW
CH = 32


def _router_body(x_ref, rw_ref, idx_ref, gate_ref):
    x = x_ref[...]
    rw = rw_ref[...]
    logits = lax.dot_general(x, rw, (((1,), (1,)), ((), ())),
                             preferred_element_type=jnp.float32)
    iota = lax.broadcasted_iota(jnp.int32, logits.shape, 1)
    m1 = jnp.max(logits, axis=1, keepdims=True)
    i1 = jnp.min(jnp.where(logits == m1, iota, E), axis=1, keepdims=True)
    l2 = jnp.where(iota == i1, -jnp.inf, logits)
    m2 = jnp.max(l2, axis=1, keepdims=True)
    i2 = jnp.min(jnp.where(l2 == m2, iota, E), axis=1, keepdims=True)
    g1 = 1.0 / (1.0 + jnp.exp(m2 - m1))
    idx_ref[...] = jnp.concatenate([i1, i2], axis=1).astype(jnp.int32)
    gate_ref[...] = jnp.concatenate([g1, 1.0 - g1], axis=1)


def _router(flat, router_w):
    return pl.pallas_call(
        _router_body,
        out_shape=(jax.ShapeDtypeStruct((T, 2), jnp.int32),
                   jax.ShapeDtypeStruct((T, 2), jnp.float32)),
    )(flat, router_w)


def _plan(idx, gate):
    e0, e1 = idx[:, 0], idx[:, 1]
    ar = jnp.arange(E, dtype=jnp.int32)
    oh = ((e0[:, None] == ar).astype(jnp.int32)
          + (e1[:, None] == ar).astype(jnp.int32))
    csum = jnp.cumsum(oh, axis=0)
    counts = csum[-1]
    before = csum - oh
    nblk = (counts + BM - 1) // BM
    blk_off = jnp.concatenate(
        [jnp.zeros((1,), nblk.dtype), jnp.cumsum(nblk)[:-1]])
    padded_off = (blk_off * BM).astype(jnp.int32)
    r0 = jnp.take_along_axis(before, e0[:, None], axis=1)[:, 0]
    r1 = jnp.take_along_axis(before, e1[:, None], axis=1)[:, 0]
    pos0 = padded_off[e0] + r0
    pos1 = padded_off[e1] + r1
    pos = jnp.stack([pos0, pos1], axis=1).reshape(-1).astype(jnp.int32)
    b = jnp.arange(NB)
    block_expert = (jnp.sum((b[:, None] >= blk_off[None, :]).astype(jnp.int32),
                            axis=1) - 1).astype(jnp.int32)
    active = (b < jnp.sum(nblk)).astype(jnp.int32)
    gate_sorted = jnp.zeros((NPAD,), jnp.float32).at[pos].set(gate.reshape(-1))
    return (pos0.astype(jnp.int32), pos1.astype(jnp.int32), pos,
            block_expert, active, gate_sorted.reshape(NPAD, 1))


def _sc_mesh():
    return plsc.VectorSubcoreMesh(core_axis_name="c", subcore_axis_name="s")


def _dispatch(flat, pos0, pos1):
    @functools.partial(
        pl.kernel,
        mesh=_sc_mesh(),
        out_type=jax.ShapeDtypeStruct((NPAD, D), jnp.float32),
        scratch_types=[
            pltpu.VMEM((TPW,), jnp.int32),
            pltpu.VMEM((TPW,), jnp.int32),
            pltpu.VMEM((TPW, D), jnp.float32),
            pltpu.SemaphoreType.DMA,
        ],
    )
    def k(flat_hbm, p0_hbm, p1_hbm, xg_hbm, i0_v, i1_v, rows_v, sem):
        wid = lax.axis_index("s") * 2 + lax.axis_index("c")
        base = wid * TPW
        pltpu.sync_copy(p0_hbm.at[pl.ds(base, TPW)], i0_v)
        pltpu.sync_copy(p1_hbm.at[pl.ds(base, TPW)], i1_v)
        pltpu.sync_copy(flat_hbm.at[pl.ds(base, TPW)], rows_v)
        pltpu.async_copy(rows_v, xg_hbm.at[i0_v], sem).wait()
        pltpu.async_copy(rows_v, xg_hbm.at[i1_v], sem).wait()

    return k(flat, pos0, pos1)


BF = 1024
NF = F // BF


def _moe_body(be_ref, act_ref, xg_ref, w1_ref, w3_ref, w2_ref, gs_ref, y_ref,
              acc_ref):
    f = pl.program_id(0)
    b = pl.program_id(1)
    active = act_ref[b] == 1
    sl = pl.ds(b * BM, BM)

    @pl.when(active)
    def _():
        xb = xg_ref[...]
        a = lax.dot_general(xb, w1_ref[0], (((1,), (1,)), ((), ())),
                            preferred_element_type=jnp.float32)
        c = lax.dot_general(xb, w3_ref[0], (((1,), (1,)), ((), ())),
                            preferred_element_type=jnp.float32)
        h = (a * jax.nn.sigmoid(a)) * c
        p = lax.dot_general(h, w2_ref[0], (((1,), (1,)), ((), ())),
                            preferred_element_type=jnp.float32)

        @pl.when(f == 0)
        def _():
            acc_ref[sl, :] = p

        @pl.when(f > 0)
        def _():
            acc_ref[sl, :] += p

    @pl.when(jnp.logical_not(active) & (f == 0))
    def _():
        acc_ref[sl, :] = jnp.zeros((BM, D), jnp.float32)

    @pl.when(f == NF - 1)
    def _():
        y_ref[...] = acc_ref[sl, :] * gs_ref[...]


def _moe(block_expert, active, xg, w1, w3, w2, gate_sorted):
    grid_spec = pltpu.PrefetchScalarGridSpec(
        num_scalar_prefetch=2,
        grid=(NF, NB),
        in_specs=[
            pl.BlockSpec((BM, D), lambda f, b, be, act: (b, 0)),
            pl.BlockSpec((1, BF, D), lambda f, b, be, act: (be[b], f, 0)),
            pl.BlockSpec((1, BF, D), lambda f, b, be, act: (be[b], f, 0)),
            pl.BlockSpec((1, D, BF), lambda f, b, be, act: (be[b], 0, f)),
            pl.BlockSpec((BM, 1), lambda f, b, be, act: (b, 0)),
        ],
        out_specs=pl.BlockSpec((BM, D), lambda f, b, be, act: (b, 0)),
        scratch_shapes=[pltpu.VMEM((NPAD, D), jnp.float32)],
    )
    return pl.pallas_call(
        _moe_body,
        grid_spec=grid_spec,
        out_shape=jax.ShapeDtypeStruct((NPAD, D), jnp.float32),
        compiler_params=pltpu.CompilerParams(
            vmem_limit_bytes=128 * 1024 * 1024),
    )(block_expert, active, xg, w1, w3, w2, gate_sorted)


def _combine(y, pos):
    @functools.partial(
        pl.kernel,
        mesh=_sc_mesh(),
        out_type=jax.ShapeDtypeStruct((T, D), jnp.float32),
        scratch_types=[
            pltpu.VMEM((2 * CH,), jnp.int32),
            pltpu.VMEM((2 * CH, D), jnp.float32),
            pltpu.VMEM((CH, D), jnp.float32),
            pltpu.SemaphoreType.DMA,
        ],
    )
    def k(y_hbm, pos_hbm, out_hbm, idx_v, rows_v, out_v, sem):
        wid = lax.axis_index("s") * 2 + lax.axis_index("c")
        for c in range(TPW // CH):
            tbase = wid * TPW + c * CH
            pltpu.sync_copy(pos_hbm.at[pl.ds(2 * tbase, 2 * CH)], idx_v)
            pltpu.async_copy(y_hbm.at[idx_v], rows_v, sem).wait()

            def body(t, _):
                def ibody(i, _):
                    s = pl.ds(i * 16, 16)
                    out_v[t, s] = rows_v[2 * t, s] + rows_v[2 * t + 1, s]
                    return 0
                lax.fori_loop(0, D // 16, ibody, 0)
                return 0
            lax.fori_loop(0, CH, body, 0)
            pltpu.sync_copy(out_v, out_hbm.at[pl.ds(tbase, CH)])

    return k(y, pos)


def kernel(x, router_w, w1, w2, w3):
    flat = x.reshape(T, D)
    idx, gate = _router(flat, router_w)
    pos0, pos1, pos, block_expert, active, gate_sorted = _plan(idx, gate)
    xg = _dispatch(flat, pos0, pos1)
    y = _moe(block_expert, active, xg, w1, w3, w2, gate_sorted)
    out = _combine(y, pos)
    return out.reshape(x.shape)
